# baseline (speedup 1.0000x reference)
import functools

import jax
import jax.numpy as jnp
import numpy as np
from jax import lax
from jax.experimental import pallas as pl
from jax.experimental.pallas import tpu as pltpu
from jax.experimental.pallas import tpu_sc as plsc

B = 16384
FEAT = 40960
HID = 512
LANES = 16

NC = 2
NS = 16
NW = NC * NS

P = 2
BCH = B // P
ROWS_PER_CALL = 2 * BCH
ROWS_PER_W = ROWS_PER_CALL // NW
CHUNK = 64
N_CHUNKS = ROWS_PER_W // CHUNK
NGRP = HID // (2 * LANES)

BS = 2048

_PERM = np.empty(HID, np.int32)
for _w in range(NGRP):
    for _i in range(LANES):
        _PERM[32 * _w + 2 * _i] = 32 * _w + _i
        _PERM[32 * _w + 2 * _i + 1] = 32 * _w + LANES + _i


def _gather_body(idx_hbm, emb_hbm, out_hbm, idx_v, fbuf, wbuf,
                 gs0, gs1, ws0, ws1):
    wid = lax.axis_index("s") * NC + lax.axis_index("c")
    base = wid * ROWS_PER_W
    gsems = (gs0, gs1)
    wsems = (ws0, ws1)

    pltpu.sync_copy(idx_hbm.at[pl.ds(base, ROWS_PER_W)], idx_v)

    def start_gather(i):
        b = i % 2
        return pltpu.async_copy(
            emb_hbm.at[idx_v.at[pl.ds(i * CHUNK, CHUNK)]], fbuf.at[b],
            gsems[b])

    def compress(b):
        src = fbuf.at[b]
        dst = wbuf.at[b]

        def row(r, carry):
            for w in range(NGRP):
                a = plsc.bitcast(src[r, pl.ds(32 * w, LANES)], jnp.int32)
                c = plsc.bitcast(src[r, pl.ds(32 * w + LANES, LANES)],
                                 jnp.int32)
                a = a + jnp.int32(0x8000)
                c = c + jnp.int32(0x8000)
                lo = lax.shift_right_logical(a, 16)
                hi = lax.bitwise_and(c, jnp.int32(-65536))
                dst[r, pl.ds(LANES * w, LANES)] = lax.bitwise_or(lo, hi)
            return carry

        lax.fori_loop(0, CHUNK, row, 0)

    gh = {}
    wh = {}
    gh[0] = start_gather(0)
    for i in range(N_CHUNKS):
        b = i % 2
        gh.pop(i).wait()
        if i + 1 < N_CHUNKS:
            gh[i + 1] = start_gather(i + 1)
        if i - 2 >= 0:
            wh.pop(i - 2).wait()
        compress(b)
        wh[i] = pltpu.async_copy(
            wbuf.at[b], out_hbm.at[pl.ds(base + i * CHUNK, CHUNK)], wsems[b])
    for i in sorted(wh):
        wh.pop(i).wait()


@functools.lru_cache(maxsize=1)
def _get_sc_gather():
    return pl.kernel(
        _gather_body,
        out_type=jax.ShapeDtypeStruct((ROWS_PER_CALL, HID // 2), jnp.int32),
        mesh=plsc.VectorSubcoreMesh(
            core_axis_name="c", subcore_axis_name="s",
            num_cores=NC, num_subcores=NS,
        ),
        scratch_types=[
            pltpu.VMEM((ROWS_PER_W,), jnp.int32),
            pltpu.VMEM((2, CHUNK, HID), jnp.float32),
            pltpu.VMEM((2, CHUNK, HID // 2), jnp.int32),
            pltpu.SemaphoreType.DMA,
            pltpu.SemaphoreType.DMA,
            pltpu.SemaphoreType.DMA,
            pltpu.SemaphoreType.DMA,
        ],
        compiler_params=pltpu.CompilerParams(needs_layout_passes=False),
    )


def _mlp_body(g_ref, w1a_ref, w1b_ref, b1_ref, w2_ref, b2_ref, w3_ref, b3_ref,
              out_ref):
    stm = jnp.clip(g_ref[0], 0.0, 1.0).astype(jnp.float32)
    nstm = jnp.clip(g_ref[1], 0.0, 1.0).astype(jnp.float32)
    h = jnp.dot(stm, w1a_ref[...], preferred_element_type=jnp.float32)
    h = h + jnp.dot(nstm, w1b_ref[...], preferred_element_type=jnp.float32)
    h = jnp.clip(h + b1_ref[0], 0.0, 1.0)
    h = jnp.clip(
        jnp.dot(h, w2_ref[...], preferred_element_type=jnp.float32) + b2_ref[0],
        0.0, 1.0)
    out_ref[...] = jnp.tanh(
        jnp.dot(h, w3_ref[...], preferred_element_type=jnp.float32) + b3_ref[0])


def _mlp(g3, W1a, W1b, b1, W2, b2, W3, b3):
    return pl.pallas_call(
        _mlp_body,
        grid=(BCH // BS,),
        in_specs=[
            pl.BlockSpec((2, BS, HID), lambda i: (0, i, 0)),
            pl.BlockSpec((HID, 128), lambda i: (0, 0)),
            pl.BlockSpec((HID, 128), lambda i: (0, 0)),
            pl.BlockSpec((1, 128), lambda i: (0, 0)),
            pl.BlockSpec((128, 32), lambda i: (0, 0)),
            pl.BlockSpec((1, 32), lambda i: (0, 0)),
            pl.BlockSpec((32, 1), lambda i: (0, 0)),
            pl.BlockSpec((1, 1), lambda i: (0, 0)),
        ],
        out_specs=pl.BlockSpec((BS, 1), lambda i: (i, 0)),
        out_shape=jax.ShapeDtypeStruct((BCH, 1), jnp.float32),
        compiler_params=pltpu.CompilerParams(
            dimension_semantics=("arbitrary",)),
    )(g3, W1a, W1b, b1, W2, b2, W3, b3)


def kernel(stm_idx, stm_off, nstm_idx, nstm_off, emb, W1, b1, W2, b2, W3, b3):
    perm = jnp.asarray(_PERM)
    W1a = W1[:HID][perm]
    W1b = W1[HID:][perm]
    b1r = b1.reshape(1, 128)
    b2r = b2.reshape(1, 32)
    b3r = b3.reshape(1, 1)
    sc_gather = _get_sc_gather()
    outs = []
    for p in range(P):
        idx_p = jnp.concatenate(
            [lax.dynamic_slice(stm_idx, (p * BCH,), (BCH,)),
             lax.dynamic_slice(nstm_idx, (p * BCH,), (BCH,))])
        gw = sc_gather(idx_p, emb)
        g = lax.bitcast_convert_type(
            gw.reshape(2, BCH, HID // 2), jnp.bfloat16).reshape(2, BCH, HID)
        outs.append(_mlp(g, W1a, W1b, b1r, W2, b2r, W3, b3r))
    return jnp.concatenate(outs, axis=0)


# speedup vs baseline: 3.1513x; 3.1513x over previous
import functools

import jax
import jax.numpy as jnp
from jax import lax
from jax.experimental import pallas as pl
from jax.experimental.pallas import tpu as pltpu
from jax.experimental.pallas import tpu_sc as plsc

B = 16384
FEAT = 40960
HID = 512

NC = 2
NS = 16
NW = NC * NS

P = 2
BCH = B // P
ROWS_PER_CALL = 2 * BCH
ROWS_PER_W = ROWS_PER_CALL // NW
CHUNK = 64
N_CHUNKS = ROWS_PER_W // CHUNK
NBUF = 3

BS = 2048


def _gather_body(idx_hbm, emb_hbm, out_hbm, idx_v, bufs,
                 gs0, gs1, gs2, ws0, ws1, ws2):
    wid = lax.axis_index("s") * NC + lax.axis_index("c")
    base = wid * ROWS_PER_W
    gsems = (gs0, gs1, gs2)
    wsems = (ws0, ws1, ws2)

    pltpu.sync_copy(idx_hbm.at[pl.ds(base, ROWS_PER_W)], idx_v)

    def start_gather(i):
        b = i % NBUF
        return pltpu.async_copy(
            emb_hbm.at[idx_v.at[pl.ds(i * CHUNK, CHUNK)]], bufs.at[b],
            gsems[b])

    gh = {}
    wh = {}
    for j in range(min(NBUF - 1, N_CHUNKS)):
        gh[j] = start_gather(j)
    for i in range(N_CHUNKS):
        b = i % NBUF
        gh[i].wait()
        wh[i] = pltpu.async_copy(
            bufs.at[b], out_hbm.at[pl.ds(base + i * CHUNK, CHUNK)], wsems[b])
        n = i + NBUF - 1
        if n < N_CHUNKS:
            if n - NBUF >= 0:
                wh.pop(n - NBUF).wait()
            gh[n] = start_gather(n)
    for i in sorted(wh):
        wh[i].wait()


@functools.lru_cache(maxsize=1)
def _get_sc_gather():
    return pl.kernel(
        _gather_body,
        out_type=jax.ShapeDtypeStruct((ROWS_PER_CALL, HID), jnp.float32),
        mesh=plsc.VectorSubcoreMesh(
            core_axis_name="c", subcore_axis_name="s",
            num_cores=NC, num_subcores=NS,
        ),
        scratch_types=[
            pltpu.VMEM((ROWS_PER_W,), jnp.int32),
            pltpu.VMEM((NBUF, CHUNK, HID), jnp.float32),
            pltpu.SemaphoreType.DMA,
            pltpu.SemaphoreType.DMA,
            pltpu.SemaphoreType.DMA,
            pltpu.SemaphoreType.DMA,
            pltpu.SemaphoreType.DMA,
            pltpu.SemaphoreType.DMA,
        ],
    )


def _mlp_body(g_ref, w1a_ref, w1b_ref, b1_ref, w2_ref, b2_ref, w3_ref, b3_ref,
              out_ref):
    stm = jnp.clip(g_ref[0], 0.0, 1.0)
    nstm = jnp.clip(g_ref[1], 0.0, 1.0)
    h = jnp.dot(stm, w1a_ref[...], preferred_element_type=jnp.float32)
    h = h + jnp.dot(nstm, w1b_ref[...], preferred_element_type=jnp.float32)
    h = jnp.clip(h + b1_ref[0], 0.0, 1.0)
    h = jnp.clip(
        jnp.dot(h, w2_ref[...], preferred_element_type=jnp.float32) + b2_ref[0],
        0.0, 1.0)
    out_ref[...] = jnp.tanh(
        jnp.dot(h, w3_ref[...], preferred_element_type=jnp.float32) + b3_ref[0])


def _mlp(g3, W1a, W1b, b1, W2, b2, W3, b3):
    return pl.pallas_call(
        _mlp_body,
        grid=(BCH // BS,),
        in_specs=[
            pl.BlockSpec((2, BS, HID), lambda i: (0, i, 0)),
            pl.BlockSpec((HID, 128), lambda i: (0, 0)),
            pl.BlockSpec((HID, 128), lambda i: (0, 0)),
            pl.BlockSpec((1, 128), lambda i: (0, 0)),
            pl.BlockSpec((128, 32), lambda i: (0, 0)),
            pl.BlockSpec((1, 32), lambda i: (0, 0)),
            pl.BlockSpec((32, 1), lambda i: (0, 0)),
            pl.BlockSpec((1, 1), lambda i: (0, 0)),
        ],
        out_specs=pl.BlockSpec((BS, 1), lambda i: (i, 0)),
        out_shape=jax.ShapeDtypeStruct((BCH, 1), jnp.float32),
        compiler_params=pltpu.CompilerParams(
            dimension_semantics=("arbitrary",)),
    )(g3, W1a, W1b, b1, W2, b2, W3, b3)


def kernel(stm_idx, stm_off, nstm_idx, nstm_off, emb, W1, b1, W2, b2, W3, b3):
    W1a = W1[:HID]
    W1b = W1[HID:]
    b1r = b1.reshape(1, 128)
    b2r = b2.reshape(1, 32)
    b3r = b3.reshape(1, 1)
    sc_gather = _get_sc_gather()
    outs = []
    for p in range(P):
        idx_p = jnp.concatenate(
            [lax.dynamic_slice(stm_idx, (p * BCH,), (BCH,)),
             lax.dynamic_slice(nstm_idx, (p * BCH,), (BCH,))])
        g = sc_gather(idx_p, emb)
        outs.append(
            _mlp(g.reshape(2, BCH, HID), W1a, W1b, b1r, W2, b2r, W3, b3r))
    return jnp.concatenate(outs, axis=0)
